# baseline (speedup 1.0000x reference)
import functools

import jax
import jax.numpy as jnp
from jax import lax
from jax.experimental import pallas as pl
from jax.experimental.pallas import tpu as pltpu
from jax.experimental.pallas import tpu_sc as plsc

SEQ = 200
EMDIM = 64
HIDDEN = 64
SLAB = 128
ROWS_PER_WORKER = 8
NUM_WORKERS = SEQ // ROWS_PER_WORKER

_info = plsc.get_sparse_core_info()
_NC = _info.num_cores
_NS = _info.num_subcores

def _bcast_lane(vec, lane):
    idx = jnp.full((16, 1), lane, jnp.int32)
    dn = lax.GatherDimensionNumbers(
        offset_dims=(), collapsed_slice_dims=(0,), start_index_map=(0,))
    return lax.gather(vec, idx, dn, (1,),
                      mode=lax.GatherScatterMode.PROMISE_IN_BOUNDS)


def _sc_body(sent_ref, table_ref, out_ref, idx_v, tiles_v, part_v, sem):
    wid = lax.axis_index("s") * _NC + lax.axis_index("c")

    @pl.when(wid < NUM_WORKERS)
    def _():
        base = wid * ROWS_PER_WORKER
        pltpu.sync_copy(sent_ref.at[pl.ds(base, ROWS_PER_WORKER)],
                        idx_v.at[pl.ds(0, ROWS_PER_WORKER)])
        idx = idx_v[...]
        lane_iota = lax.iota(jnp.int32, 16)
        valid = lane_iota < ROWS_PER_WORKER
        aligned = jnp.where(valid, idx & jnp.int32(~(SLAB - 1)), 0)
        cols = jnp.where(valid, idx & (SLAB - 1), 0)
        copies = []
        for r in range(ROWS_PER_WORKER):
            a_r = pl.multiple_of(
                jnp.max(jnp.where(lane_iota == r, aligned, 0)), SLAB)
            copies.append(
                pltpu.async_copy(table_ref.at[:, pl.ds(a_r, SLAB)],
                                 tiles_v.at[r], sem))
        for cp in copies:
            cp.wait()
        accs = []
        for c in range(EMDIM // 16):
            accs.append(jnp.zeros((16,), jnp.float32))
        for r in range(ROWS_PER_WORKER):
            col_sel = _bcast_lane(cols, r)
            tile_sel = jnp.full((16,), r, jnp.int32)
            for c in range(EMDIM // 16):
                vals = plsc.load_gather(
                    tiles_v, [tile_sel, lane_iota + c * 16, col_sel])
                accs[c] = accs[c] + vals
        for c in range(EMDIM // 16):
            part_v[pl.ds(c * 16, 16)] = accs[c]
        pltpu.sync_copy(part_v, out_ref.at[wid])


@functools.partial(
    pl.kernel,
    out_type=jax.ShapeDtypeStruct((NUM_WORKERS, EMDIM), jnp.float32),
    mesh=plsc.VectorSubcoreMesh(core_axis_name="c", subcore_axis_name="s"),
    scratch_types=[
        pltpu.VMEM((16,), jnp.int32),
        pltpu.VMEM((ROWS_PER_WORKER, EMDIM, SLAB), jnp.float32),
        pltpu.VMEM((EMDIM,), jnp.float32),
        pltpu.SemaphoreType.DMA,
    ],
    compiler_params=pltpu.CompilerParams(
        use_tc_tiling_on_sc=True, needs_layout_passes=False,
        disable_bounds_checks=True, skip_device_barrier=True),
)
def _sc_gather_sum(sent_ref, table_ref, out_ref, idx_v, tiles_v, part_v, sem):
    _sc_body(sent_ref, table_ref, out_ref, idx_v, tiles_v, part_v, sem)


def _tc_head(p_ref, w_ref, b_ref, o_ref):
    x = jnp.sum(p_ref[...], axis=0, keepdims=True) * (1.0 / SEQ)
    y = lax.dot_general(
        x, w_ref[...], (((1,), (1,)), ((), ())),
        preferred_element_type=jnp.float32,
    )
    o_ref[...] = jnp.tanh(y + b_ref[...])


def kernel(sentence, table, W, b):
    partials = _sc_gather_sum(sentence.astype(jnp.int32), table.T)
    out = pl.pallas_call(
        _tc_head,
        out_shape=jax.ShapeDtypeStruct((1, HIDDEN), jnp.float32),
        compiler_params=pltpu.CompilerParams(skip_device_barrier=True),
    )(partials, W, b.reshape(1, HIDDEN))
    return out


# speedup vs baseline: 1.0098x; 1.0098x over previous
import functools

import jax
import jax.numpy as jnp
from jax import lax
from jax.experimental import pallas as pl
from jax.experimental.pallas import tpu as pltpu
from jax.experimental.pallas import tpu_sc as plsc

SEQ = 200
EMDIM = 64
HIDDEN = 64
SLAB = 128
ROWS_PER_WORKER = 8
NUM_WORKERS = SEQ // ROWS_PER_WORKER

_info = plsc.get_sparse_core_info()
_NC = _info.num_cores
_NS = _info.num_subcores

def _bcast_lane(vec, lane):
    idx = jnp.full((16, 1), lane, jnp.int32)
    dn = lax.GatherDimensionNumbers(
        offset_dims=(), collapsed_slice_dims=(0,), start_index_map=(0,))
    return lax.gather(vec, idx, dn, (1,),
                      mode=lax.GatherScatterMode.PROMISE_IN_BOUNDS)


def _sc_body(sent_ref, table_ref, out_ref, idx_v, tiles_v, part_v, sem):
    wid = lax.axis_index("s") * _NC + lax.axis_index("c")

    @pl.when(wid < NUM_WORKERS)
    def _():
        base = wid * ROWS_PER_WORKER
        pltpu.sync_copy(sent_ref.at[pl.ds(base, ROWS_PER_WORKER)],
                        idx_v.at[pl.ds(0, ROWS_PER_WORKER)])
        idx = idx_v[...]
        lane_iota = lax.iota(jnp.int32, 16)
        valid = lane_iota < ROWS_PER_WORKER
        aligned = jnp.where(valid, idx & jnp.int32(~(SLAB - 1)), 0)
        cols = jnp.where(valid, idx & (SLAB - 1), 0)
        def _issue(r, carry):
            a_r = pl.multiple_of(
                jnp.max(jnp.where(lane_iota == r, aligned, 0)), SLAB)
            pltpu.async_copy(table_ref.at[:, pl.ds(a_r, SLAB)],
                             tiles_v.at[r], sem)
            return carry

        lax.fori_loop(0, ROWS_PER_WORKER, _issue, 0)

        def _drain(r, carry):
            pltpu.make_async_copy(table_ref.at[:, pl.ds(0, SLAB)],
                                  tiles_v.at[r], sem).wait()
            return carry

        lax.fori_loop(0, ROWS_PER_WORKER, _drain, 0)

        def _accum(r, accs):
            col_sel = _bcast_lane(cols, r)
            tile_sel = jnp.full((16,), r, jnp.int32)
            return tuple(
                accs[c] + plsc.load_gather(
                    tiles_v, [tile_sel, lane_iota + c * 16, col_sel])
                for c in range(EMDIM // 16))

        zeros = tuple(jnp.zeros((16,), jnp.float32)
                      for _ in range(EMDIM // 16))
        accs = lax.fori_loop(0, ROWS_PER_WORKER, _accum, zeros)
        for c in range(EMDIM // 16):
            part_v[pl.ds(c * 16, 16)] = accs[c]
        pltpu.sync_copy(part_v, out_ref.at[wid])


@functools.partial(
    pl.kernel,
    out_type=jax.ShapeDtypeStruct((NUM_WORKERS, EMDIM), jnp.float32),
    mesh=plsc.VectorSubcoreMesh(core_axis_name="c", subcore_axis_name="s"),
    scratch_types=[
        pltpu.VMEM((16,), jnp.int32),
        pltpu.VMEM((ROWS_PER_WORKER, EMDIM, SLAB), jnp.float32),
        pltpu.VMEM((EMDIM,), jnp.float32),
        pltpu.SemaphoreType.DMA,
    ],
    compiler_params=pltpu.CompilerParams(
        use_tc_tiling_on_sc=True, needs_layout_passes=False,
        disable_bounds_checks=True, skip_device_barrier=True),
)
def _sc_gather_sum(sent_ref, table_ref, out_ref, idx_v, tiles_v, part_v, sem):
    _sc_body(sent_ref, table_ref, out_ref, idx_v, tiles_v, part_v, sem)


def _tc_head(p_ref, w_ref, b_ref, o_ref):
    x = jnp.sum(p_ref[...], axis=0, keepdims=True) * (1.0 / SEQ)
    y = lax.dot_general(
        x, w_ref[...], (((1,), (1,)), ((), ())),
        preferred_element_type=jnp.float32,
    )
    o_ref[...] = jnp.tanh(y + b_ref[...])


def kernel(sentence, table, W, b):
    partials = _sc_gather_sum(sentence.astype(jnp.int32), table.T)
    out = pl.pallas_call(
        _tc_head,
        out_shape=jax.ShapeDtypeStruct((1, HIDDEN), jnp.float32),
        compiler_params=pltpu.CompilerParams(skip_device_barrier=True),
    )(partials, W, b.reshape(1, HIDDEN))
    return out
